# baseline (speedup 1.0000x reference)
import functools

import jax
import jax.numpy as jnp
from jax import lax
from jax.experimental import pallas as pl
from jax.experimental.pallas import tpu as pltpu
from jax.experimental.pallas import tpu_sc as plsc


def kernel(id_indices, embeddings):
    (B,) = id_indices.shape
    V, D = embeddings.shape

    info = plsc.get_sparse_core_info()
    NC, NS = info.num_cores, info.num_subcores
    NW = NC * NS
    assert B % (8 * NW) == 0
    b_per_w = B // NW

    mesh = plsc.VectorSubcoreMesh(core_axis_name="c", subcore_axis_name="s")

    @functools.partial(
        pl.kernel,
        mesh=mesh,
        out_type=jax.ShapeDtypeStruct((B, D), jnp.float32),
        compiler_params=pltpu.CompilerParams(use_tc_tiling_on_sc=False),
        scratch_types=[
            pltpu.VMEM((b_per_w,), jnp.int32),
            pltpu.VMEM((b_per_w, D), jnp.float32),
            pltpu.SemaphoreType.DMA,
        ],
    )
    def gather_kernel(table_hbm, idx_hbm, out_hbm, idx_v, rows_v, sem):
        wid = lax.axis_index("s") * NC + lax.axis_index("c")
        base = wid * b_per_w
        pltpu.sync_copy(idx_hbm.at[pl.ds(base, b_per_w)], idx_v)
        pltpu.async_copy(table_hbm.at[idx_v], rows_v, sem).wait()
        pltpu.sync_copy(rows_v, out_hbm.at[pl.ds(base, b_per_w)])

    return gather_kernel(embeddings, id_indices.astype(jnp.int32))


# speedup vs baseline: 1.0003x; 1.0003x over previous
import functools

import jax
import jax.numpy as jnp
from jax import lax
from jax.experimental import pallas as pl
from jax.experimental.pallas import tpu as pltpu
from jax.experimental.pallas import tpu_sc as plsc


def kernel(id_indices, embeddings):
    (B,) = id_indices.shape
    V, D = embeddings.shape

    info = plsc.get_sparse_core_info()
    NC, NS = info.num_cores, info.num_subcores
    NW = NC * NS
    assert B % (8 * NW) == 0
    b_per_w = B // NW

    mesh = plsc.VectorSubcoreMesh(core_axis_name="c", subcore_axis_name="s")

    @functools.partial(
        pl.kernel,
        mesh=mesh,
        out_type=jax.ShapeDtypeStruct((B, D), jnp.float32),
        compiler_params=pltpu.CompilerParams(
            use_tc_tiling_on_sc=False,
            skip_device_barrier=True,
            disable_bounds_checks=True,
            disable_semaphore_checks=True,
        ),
        scratch_types=[
            pltpu.VMEM((b_per_w,), jnp.int32),
            pltpu.VMEM((b_per_w, D), jnp.float32),
            pltpu.SemaphoreType.DMA,
        ],
    )
    def gather_kernel(table_hbm, idx_hbm, out_hbm, idx_v, rows_v, sem):
        wid = lax.axis_index("s") * NC + lax.axis_index("c")
        base = wid * b_per_w
        pltpu.sync_copy(idx_hbm.at[pl.ds(base, b_per_w)], idx_v)
        pltpu.async_copy(table_hbm.at[idx_v], rows_v, sem).wait()
        pltpu.sync_copy(rows_v, out_hbm.at[pl.ds(base, b_per_w)])

    return gather_kernel(embeddings, id_indices.astype(jnp.int32))


# speedup vs baseline: 1.4246x; 1.4241x over previous
import functools

import jax
import jax.numpy as jnp
from jax import lax
from jax.experimental import pallas as pl
from jax.experimental.pallas import tpu as pltpu
from jax.experimental.pallas import tpu_sc as plsc


def kernel(id_indices, embeddings):
    (B,) = id_indices.shape
    V, D = embeddings.shape

    info = plsc.get_sparse_core_info()
    NC, NS, L = info.num_cores, info.num_subcores, info.num_lanes
    NW = NC * NS
    b_per_w = B // NW
    n_chunk = 16
    c_sz = b_per_w // n_chunk

    mesh = plsc.VectorSubcoreMesh(core_axis_name="c", subcore_axis_name="s")

    @functools.partial(
        pl.kernel,
        mesh=mesh,
        out_type=jax.ShapeDtypeStruct((B, D), jnp.float32),
        compiler_params=pltpu.CompilerParams(
            use_tc_tiling_on_sc=True, needs_layout_passes=False
        ),
        scratch_types=[
            pltpu.VMEM((b_per_w + L,), jnp.int32),
            pltpu.VMEM((b_per_w + L,), jnp.int32),
            pltpu.VMEM((c_sz * 8, D), jnp.float32),
            pltpu.VMEM((b_per_w, D), jnp.float32),
            pltpu.SemaphoreType.DMA,
        ],
    )
    def gather_kernel(tbl, idx_hbm, out_hbm, idx_v, r8_v, gat2, out_v, sem):
        c = lax.axis_index("c")
        s = lax.axis_index("s")
        w = s * NC + c
        base = w * b_per_w

        pltpu.sync_copy(idx_hbm.at[pl.ds(base, b_per_w)], idx_v.at[pl.ds(0, b_per_w)])

        def to_blocks(k, carry):
            v = idx_v[pl.ds(k * L, L)]
            r8_v[pl.ds(k * L, L)] = (v >> 3) * 8
            return carry

        lax.fori_loop(0, b_per_w // L, to_blocks, 0)

        lane = lax.iota(jnp.int32, L)

        for chunk in range(n_chunk):
            def issue(j, carry):
                r8 = pl.multiple_of(r8_v[pl.ds(chunk * c_sz + j, L)][0], 8)
                pltpu.make_async_copy(
                    tbl.at[pl.ds(r8, 8)],
                    gat2.at[pl.ds(j * 8, 8)],
                    sem,
                ).start()
                return carry

            lax.fori_loop(0, c_sz, issue, 0)

            def drain(j, carry):
                pltpu.make_async_copy(
                    tbl.at[pl.ds(0, 8)], gat2.at[pl.ds(j * 8, 8)], sem
                ).wait()
                return carry

            lax.fori_loop(0, c_sz, drain, 0)

            def pick(g, carry):
                iabs = chunk * c_sz + g * L
                sub = idx_v[pl.ds(iabs, L)] & 7
                row16 = (lane + g * L) * 8 + sub
                j_abs = lane + iabs
                for l in range(D):
                    lsplat = jnp.full((L,), l, jnp.int32)
                    vals = plsc.load_gather(gat2, [row16, lsplat])
                    plsc.store_scatter(out_v, [j_abs, lsplat], vals)
                return carry

            lax.fori_loop(0, c_sz // L, pick, 0)

        pltpu.sync_copy(out_v, out_hbm.at[pl.ds(base, b_per_w)])

    emb_t = embeddings[: (V // 8) * 8]
    return gather_kernel(emb_t, id_indices.astype(jnp.int32))


# speedup vs baseline: 1.4575x; 1.0231x over previous
import functools

import jax
import jax.numpy as jnp
from jax import lax
from jax.experimental import pallas as pl
from jax.experimental.pallas import tpu as pltpu
from jax.experimental.pallas import tpu_sc as plsc


def kernel(id_indices, embeddings):
    (B,) = id_indices.shape
    V, D = embeddings.shape

    info = plsc.get_sparse_core_info()
    NC, NS, L = info.num_cores, info.num_subcores, info.num_lanes
    NW = NC * NS
    b_per_w = B // NW
    n_chunk = 16
    c_sz = b_per_w // n_chunk

    mesh = plsc.VectorSubcoreMesh(core_axis_name="c", subcore_axis_name="s")

    @functools.partial(
        pl.kernel,
        mesh=mesh,
        out_type=jax.ShapeDtypeStruct((B, D), jnp.float32),
        compiler_params=pltpu.CompilerParams(use_tc_tiling_on_sc=True),
        scratch_types=[
            pltpu.VMEM((b_per_w + L,), jnp.int32),
            pltpu.VMEM((b_per_w + L,), jnp.int32),
            pltpu.VMEM((c_sz * 8, D), jnp.float32),
            pltpu.VMEM((b_per_w, D), jnp.float32),
            pltpu.SemaphoreType.DMA,
        ],
    )
    def gather_kernel(tbl, idx_hbm, out_hbm, idx_v, r8_v, gat2, out_v, sem):
        c = lax.axis_index("c")
        s = lax.axis_index("s")
        w = s * NC + c
        base = w * b_per_w

        pltpu.sync_copy(idx_hbm.at[pl.ds(base, b_per_w)], idx_v.at[pl.ds(0, b_per_w)])

        def to_blocks(k, carry):
            v = idx_v[pl.ds(k * L, L)]
            r8_v[pl.ds(k * L, L)] = (v >> 3) * 8
            return carry

        lax.fori_loop(0, b_per_w // L, to_blocks, 0)

        lane = lax.iota(jnp.int32, L)

        for chunk in range(n_chunk):
            def issue(j, carry):
                r8 = pl.multiple_of(r8_v[pl.ds(chunk * c_sz + j, L)][0], 8)
                pltpu.make_async_copy(
                    tbl.at[pl.ds(r8, 8)],
                    gat2.at[pl.ds(j * 8, 8)],
                    sem,
                ).start()
                return carry

            lax.fori_loop(0, c_sz, issue, 0)

            def drain(j, carry):
                pltpu.make_async_copy(
                    tbl.at[pl.ds(0, 8)], gat2.at[pl.ds(j * 8, 8)], sem
                ).wait()
                return carry

            lax.fori_loop(0, c_sz, drain, 0)

            def pick(j, carry):
                jj = chunk * c_sz + j
                sub = idx_v[pl.ds(jj, L)][0] & 7
                row = j * 8 + sub
                for h in range(D // L):
                    out_v[jj, pl.ds(h * L, L)] = gat2[row, pl.ds(h * L, L)]
                return carry

            lax.fori_loop(0, c_sz, pick, 0)

        pltpu.sync_copy(out_v, out_hbm.at[pl.ds(base, b_per_w)])

    emb_t = embeddings[: (V // 8) * 8]
    return gather_kernel(emb_t, id_indices.astype(jnp.int32))
